# baseline (speedup 1.0000x reference)
import functools

import jax
import jax.numpy as jnp
from jax import lax
from jax.experimental import pallas as pl
from jax.experimental.pallas import tpu as pltpu
from jax.experimental.pallas import tpu_sc as plsc

D = 32
NC, NS = 2, 16
NW = NC * NS
BB = 4
SPLITS = ((0, 120), (120, 80))


@functools.lru_cache(maxsize=None)
def _make_gather(B: int, H: int):
    BW = B // NW
    NB = BW // BB
    assert NB % 2 == 0 and NB >= 4
    mesh = plsc.VectorSubcoreMesh(core_axis_name="c", subcore_axis_name="s")

    @functools.partial(
        pl.kernel,
        out_type=jax.ShapeDtypeStruct((B, H, D), jnp.float32),
        mesh=mesh,
        scratch_types=[
            pltpu.VMEM((2, BB, H), jnp.int32),
            pltpu.VMEM((2, BB, H, D), jnp.float32),
            pltpu.SemaphoreType.DMA,
            pltpu.SemaphoreType.DMA,
        ],
        compiler_params=pltpu.CompilerParams(use_tc_tiling_on_sc=False),
    )
    def body(idx_hbm, tab_hbm, out_hbm, idx_v, rows_v, gsem, osem):
        wid = lax.axis_index("s") * NC + lax.axis_index("c")
        b0 = wid * BW

        def fire_gathers(g, buf):
            pltpu.sync_copy(idx_hbm.at[pl.ds(b0 + g * BB, BB)], idx_v.at[buf])
            for bb in range(BB):
                for off, n in SPLITS:
                    pltpu.async_copy(
                        tab_hbm.at[idx_v.at[buf, bb, pl.ds(off, n)]],
                        rows_v.at[buf, bb, pl.ds(off, n)],
                        gsem,
                    )

        def drain_gathers(buf):
            for bb in range(BB):
                for off, n in SPLITS:
                    pltpu.make_async_copy(
                        tab_hbm.at[idx_v.at[buf, bb, pl.ds(off, n)]],
                        rows_v.at[buf, bb, pl.ds(off, n)],
                        gsem,
                    ).wait()

        def store(g, buf):
            pltpu.async_copy(rows_v.at[buf],
                             out_hbm.at[pl.ds(b0 + g * BB, BB)],
                             osem).wait()

        fire_gathers(0, 0)
        fire_gathers(1, 1)

        @pl.loop(0, NB - 2, step=2)
        def _pair(g0):
            for buf in range(2):
                g = g0 + buf
                drain_gathers(buf)
                store(g, buf)
                fire_gathers(g + 2, buf)

        for buf in range(2):
            drain_gathers(buf)
            store(NB - 2 + buf, buf)

    return body


def kernel(x, E):
    B, H = x.shape
    return _make_gather(B, H)(x.astype(jnp.int32), E)
